# baseline (speedup 1.0000x reference)
import functools

import jax
import jax.numpy as jnp
from jax import lax
from jax.experimental import pallas as pl
from jax.experimental.pallas import tpu as pltpu
from jax.experimental.pallas import tpu_sc as plsc

_L = 128
_NB = 5


@functools.cache
def _build(seq: int, dim: int):
    info = plsc.get_sparse_core_info()
    nc, ns = info.num_cores, info.num_subcores
    nw = nc * ns
    nd8 = dim // 8
    nk = dim // 16
    assert seq % _NB == 0 and seq > _NB
    mesh = plsc.VectorSubcoreMesh(core_axis_name="c", subcore_axis_name="s")

    @functools.partial(
        pl.kernel,
        mesh=mesh,
        compiler_params=pltpu.CompilerParams(
            use_tc_tiling_on_sc=False, needs_layout_passes=False
        ),
        out_type=jax.ShapeDtypeStruct((seq, nd8, nw, 8 * _L), jnp.float32),
        scratch_types=[
            pltpu.VMEM((seq, _L), jnp.int32),
            pltpu.VMEM((_NB, _L, dim), jnp.float32),
            pltpu.VMEM((_NB, 8, nd8 * _L), jnp.float32),
            [pltpu.SemaphoreType.DMA] * _NB,
            [pltpu.SemaphoreType.DMA] * _NB,
        ],
    )
    def k(tid_hbm, table_hbm, out_hbm, idx_v, raw, trn, gsems, wsems):
        w = lax.axis_index("s") * nc + lax.axis_index("c")
        pltpu.sync_copy(tid_hbm.at[:, pl.ds(w * _L, _L)], idx_v)

        iota16 = lax.iota(jnp.int32, 16)

        def gather(s, p):
            pltpu.async_copy(table_hbm.at[idx_v.at[s]], raw.at[p], gsems[p])

        for j in range(_NB):
            gather(j, j)

        def body(g, _):
            for b in range(_NB):
                s = g * _NB + b

                pltpu.make_async_copy(
                    table_hbm.at[idx_v.at[s]], raw.at[b], gsems[b]
                ).wait()

                @pl.when(s >= _NB)
                def _():
                    pltpu.make_async_copy(
                        trn.at[b], out_hbm.at[0, :, 0], wsems[b]
                    ).wait()

                raw_b = raw.at[b]
                trn_b = trn.at[b]

                @plsc.parallel_loop(0, dim, unroll=4)
                def dloop(d0):
                    dcols = (d0 + iota16) & (dim - 1)
                    r_idx = lax.shift_right_logical(dcols, 3)
                    cbase = (dcols & 7) * _L
                    for m in range(_L // 16):
                        rows = iota16 + 16 * m
                        v = plsc.load_gather(raw_b, [rows, dcols])
                        plsc.store_scatter(trn_b, [r_idx, cbase + rows], v)

                @pl.when(s + _NB < seq)
                def _():
                    gather(s + _NB, b)

                pltpu.async_copy(trn.at[b], out_hbm.at[s, :, w], wsems[b])
            return 0

        lax.fori_loop(0, seq // _NB, body, 0, unroll=False)
        for b in range(_NB):
            pltpu.make_async_copy(
                trn.at[b], out_hbm.at[0, :, 0], wsems[b]
            ).wait()

    return k, nw


def kernel(token_ids, weights):
    bsz, seq = token_ids.shape
    dim = weights.shape[1]
    k, nw = _build(seq, dim)
    wpad = jnp.pad(weights, ((0, 0), (0, 128 - dim))).reshape(-1, dim)
    tid_t = token_ids.T.astype(jnp.int32) * (128 // dim)
    out4 = k(tid_t, wpad)
    out5 = out4.reshape(seq, dim // 8, nw, 8, _L)
    return out5.transpose(2, 4, 0, 1, 3).reshape(bsz, seq, dim)


# speedup vs baseline: 1.2521x; 1.2521x over previous
import functools

import jax
import jax.numpy as jnp
from jax import lax
from jax.experimental import pallas as pl
from jax.experimental.pallas import tpu as pltpu
from jax.experimental.pallas import tpu_sc as plsc

_L = 128
_NB = 5


@functools.cache
def _build(seq: int, dim: int):
    info = plsc.get_sparse_core_info()
    nc, ns = info.num_cores, info.num_subcores
    nw = nc * ns
    nd8 = dim // 8
    nk = dim // 16
    assert seq % _NB == 0 and seq > _NB
    mesh = plsc.VectorSubcoreMesh(core_axis_name="c", subcore_axis_name="s")

    @functools.partial(
        pl.kernel,
        mesh=mesh,
        compiler_params=pltpu.CompilerParams(
            use_tc_tiling_on_sc=False, needs_layout_passes=False
        ),
        out_type=jax.ShapeDtypeStruct((seq, nd8, nw, 8 * _L), jnp.float32),
        scratch_types=[
            pltpu.VMEM((seq, _L), jnp.int32),
            pltpu.VMEM((_NB, _L, dim), jnp.float32),
            pltpu.VMEM((_NB, dim * _L), jnp.float32),
            [pltpu.SemaphoreType.DMA] * _NB,
            [pltpu.SemaphoreType.DMA] * _NB,
        ],
    )
    def k(tid_hbm, table_hbm, out_hbm, idx_v, raw, trn, gsems, wsems):
        w = lax.axis_index("s") * nc + lax.axis_index("c")
        pltpu.sync_copy(tid_hbm.at[:, pl.ds(w * _L, _L)], idx_v)

        iota16 = lax.iota(jnp.int32, 16)

        def gather(s, p):
            pltpu.async_copy(table_hbm.at[idx_v.at[s]], raw.at[p], gsems[p])

        for j in range(_NB):
            gather(j, j)

        def body(g, _):
            for b in range(_NB):
                s = g * _NB + b

                pltpu.make_async_copy(
                    table_hbm.at[idx_v.at[s]], raw.at[b], gsems[b]
                ).wait()

                @pl.when(s >= _NB)
                def _():
                    for _i in range(nd8):
                        pltpu.make_async_copy(
                            trn.at[b, pl.ds(0, 8 * _L)],
                            out_hbm.at[0, 0, 0],
                            wsems[b],
                        ).wait()

                raw_b = raw.at[b]
                trn_b = trn.at[b]

                @plsc.parallel_loop(0, dim, unroll=8)
                def dloop(d0):
                    dcols = (d0 + iota16) & (dim - 1)
                    dbase = dcols * _L
                    for m in range(_L // 16):
                        rows = iota16 + 16 * m
                        v = plsc.load_gather(raw_b, [rows, dcols])
                        plsc.store_scatter(trn_b, [dbase + rows], v)

                @pl.when(s + _NB < seq)
                def _():
                    gather(s + _NB, b)

                for i in range(nd8):
                    pltpu.async_copy(
                        trn.at[b, pl.ds(8 * _L * i, 8 * _L)],
                        out_hbm.at[s, i, w],
                        wsems[b],
                    )
            return 0

        lax.fori_loop(0, seq // _NB, body, 0, unroll=False)
        for b in range(_NB):
            for _i in range(nd8):
                pltpu.make_async_copy(
                    trn.at[b, pl.ds(0, 8 * _L)], out_hbm.at[0, 0, 0], wsems[b]
                ).wait()

    return k, nw


def kernel(token_ids, weights):
    bsz, seq = token_ids.shape
    dim = weights.shape[1]
    k, nw = _build(seq, dim)
    wpad = jnp.pad(weights, ((0, 0), (0, 128 - dim))).reshape(-1, dim)
    tid_t = token_ids.T.astype(jnp.int32) * (128 // dim)
    out4 = k(tid_t, wpad)
    out5 = out4.reshape(seq, dim // 8, nw, 8, _L)
    return out5.transpose(2, 4, 0, 1, 3).reshape(bsz, seq, dim)
